# baseline (speedup 1.0000x reference)
import functools

import jax
import jax.numpy as jnp
from jax import lax
from jax.experimental import pallas as pl
from jax.experimental.pallas import tpu as pltpu
from jax.experimental.pallas import tpu_sc as plsc

BATCH = 4096
NUM_FIELD = 26
EMB = 32
LANES = 16

NUM_CORES = 2
NUM_SUBCORES = 16
NUM_WORKERS = NUM_CORES * NUM_SUBCORES
BPW = BATCH // NUM_WORKERS
NIDX = BPW * NUM_FIELD
NCHUNK = NIDX // 128
PAD = NIDX + LANES

_mesh = plsc.VectorSubcoreMesh(core_axis_name="c", subcore_axis_name="s")

_T_BLK_C = 8192
_T_BLK_R = _T_BLK_C // 4


_SC_T_COLS = 800
_SC_T_CHUNKS = 100000 // _SC_T_COLS


@functools.partial(
    pl.kernel,
    mesh=_mesh,
    out_type=jax.ShapeDtypeStruct((100000, EMB), jnp.float32),
    scratch_types=[
        pltpu.VMEM((EMB, _SC_T_COLS), jnp.float32),
        pltpu.VMEM((_SC_T_COLS, EMB), jnp.float32),
    ],
    compiler_params=pltpu.CompilerParams(
        needs_layout_passes=False, use_tc_tiling_on_sc=False),
)
def _sc_transpose(src_hbm, out_hbm, in_v, out_v):
    wid = lax.axis_index("s") * NUM_CORES + lax.axis_index("c")
    iota = lax.iota(jnp.int32, LANES)

    def do_chunk(ch):
        c0 = ch * _SC_T_COLS
        pltpu.sync_copy(src_hbm.at[:, pl.ds(c0, _SC_T_COLS)], in_v)

        def rows_body(g, carry):
            r0 = g * LANES
            for k in range(LANES):
                col = jnp.full((LANES,), r0 + k, jnp.int32)
                out_v[r0 + k, pl.ds(0, LANES)] = plsc.load_gather(
                    in_v, [iota, col])
                out_v[r0 + k, pl.ds(LANES, LANES)] = plsc.load_gather(
                    in_v, [iota + LANES, col])
            return carry

        lax.fori_loop(0, _SC_T_COLS // LANES, rows_body, 0)
        pltpu.sync_copy(out_v, out_hbm.at[pl.ds(c0, _SC_T_COLS), :])

    def chunk_loop(j, carry):
        ch = wid + NUM_WORKERS * j

        @pl.when(ch < _SC_T_CHUNKS)
        def _():
            do_chunk(ch)

        return carry

    lax.fori_loop(0, 4, chunk_loop, 0)


def _transpose_body(in_ref, out_ref):
    x = in_ref[...]
    eye = jnp.eye(EMB, dtype=jnp.float32)
    y = lax.dot_general(x, eye, (((0,), (0,)), ((), ())),
                        preferred_element_type=jnp.float32)
    y3 = y.reshape(_T_BLK_R, 4, EMB)
    for q in range(4):
        out_ref[:, q * EMB:(q + 1) * EMB] = y3[:, q, :]


def _emb_to_lin128(femb_t):
    grid = (100000 + _T_BLK_C - 1) // _T_BLK_C
    return pl.pallas_call(
        _transpose_body,
        grid=(grid,),
        in_specs=[pl.BlockSpec((32, _T_BLK_C), lambda j: (0, j))],
        out_specs=pl.BlockSpec((_T_BLK_R, 128), lambda j: (j, 0)),
        out_shape=jax.ShapeDtypeStruct((25000, 128), jnp.float32),
    )(femb_t)


@functools.partial(
    pl.kernel,
    mesh=_mesh,
    out_type=jax.ShapeDtypeStruct((BATCH,), jnp.float32),
    scratch_types=[
        pltpu.VMEM((NIDX,), jnp.int32),
        pltpu.VMEM((PAD,), jnp.float32),
        pltpu.VMEM((PAD,), jnp.float32),
        pltpu.VMEM((NIDX, EMB), jnp.float32),
        pltpu.VMEM((BPW, LANES), jnp.float32),
        pltpu.VMEM((BPW,), jnp.float32),
        pltpu.VMEM((LANES,), jnp.float32),
        pltpu.SemaphoreType.DMA,
    ],
    compiler_params=pltpu.CompilerParams(
        needs_layout_passes=False, use_tc_tiling_on_sc=False),
)
def _fm_sc(emb_hbm, fw_hbm, idx_hbm, fv_hbm, bias_hbm, out_hbm,
           idx_v, fv_v, fw_v, rows_v, vsum_v, out_v, bias_v, sem):
    wid = lax.axis_index("s") * NUM_CORES + lax.axis_index("c")
    base = wid * NIDX

    pltpu.sync_copy(idx_hbm.at[pl.ds(base, NIDX)], idx_v)
    pltpu.sync_copy(fv_hbm.at[pl.ds(base, NIDX)], fv_v.at[pl.ds(0, NIDX)])
    pltpu.sync_copy(bias_hbm, bias_v)

    copies = []
    for c in range(NCHUNK):
        sl = pl.ds(c * 128, 128)
        copies.append(
            pltpu.async_copy(emb_hbm.at[idx_v.at[sl]], rows_v.at[sl], sem))
        copies.append(
            pltpu.async_copy(fw_hbm.at[idx_v.at[sl]], fw_v.at[sl], sem))
    for cp in copies:
        cp.wait()

    iota = lax.iota(jnp.int32, LANES)
    m10 = iota < (NUM_FIELD - LANES)
    zeros = jnp.zeros((LANES,), jnp.float32)

    def row_body(b, carry):
        j0 = b * NUM_FIELD
        acc0 = acc1 = sq0 = sq1 = zeros
        fvr0 = fv_v[pl.ds(j0, LANES)]
        fvr1 = fv_v[pl.ds(j0 + LANES, LANES)]
        for f in range(NUM_FIELD):
            e0 = rows_v[j0 + f, pl.ds(0, LANES)]
            e1 = rows_v[j0 + f, pl.ds(LANES, LANES)]
            fvs = fvr0[f] if f < LANES else fvr1[f - LANES]
            t0 = e0 * fvs
            t1 = e1 * fvs
            acc0 = acc0 + t0
            acc1 = acc1 + t1
            sq0 = sq0 + t0 * t0
            sq1 = sq1 + t1 * t1
        v = (acc0 * acc0 + acc1 * acc1 - sq0 - sq1) * 0.5
        i0 = j0 + iota
        i1 = i0 + LANES
        p0 = plsc.load_gather(fv_v, [i0]) * plsc.load_gather(fw_v, [i0])
        p1 = plsc.load_gather(fv_v, [i1]) * plsc.load_gather(fw_v, [i1])
        v = v + p0 + jnp.where(m10, p1, 0.0)
        vsum_v[b, pl.ds(0, LANES)] = v
        return carry

    lax.fori_loop(0, BPW, row_body, 0)

    bias_vec = bias_v[...]

    def red_body(g, carry):
        rb = g * LANES + iota
        y = zeros
        for k in range(LANES):
            col = jnp.full((LANES,), k, jnp.int32)
            y = y + plsc.load_gather(vsum_v, [rb, col])
        x = y + bias_vec
        out_v[pl.ds(g * LANES, LANES)] = 1.0 / (1.0 + jnp.exp(-x))
        return carry

    lax.fori_loop(0, BPW // LANES, red_body, 0)

    pltpu.sync_copy(out_v, out_hbm.at[pl.ds(wid * BPW, BPW)])


def kernel(feat_index, feat_value, first_weights, feat_embeddings, bias):
    idx = feat_index.astype(jnp.int32).reshape(-1)
    fv = feat_value.astype(jnp.float32).reshape(-1)
    fw = first_weights.astype(jnp.float32).reshape(-1)
    bias_arr = jnp.full((LANES,), bias, jnp.float32)
    emb_lin = _sc_transpose(feat_embeddings.T)
    out = _fm_sc(emb_lin, fw, idx, fv, bias_arr)
    return out.reshape(BATCH, 1)


# speedup vs baseline: 1.6469x; 1.6469x over previous
import functools

import jax
import jax.numpy as jnp
from jax import lax
from jax.experimental import pallas as pl
from jax.experimental.pallas import tpu as pltpu
from jax.experimental.pallas import tpu_sc as plsc

BATCH = 4096
NUM_FIELD = 26
EMB = 32
LANES = 16

NUM_CORES = 2
NUM_SUBCORES = 16
NUM_WORKERS = NUM_CORES * NUM_SUBCORES
BPW = BATCH // NUM_WORKERS
NIDX = BPW * NUM_FIELD
NCHUNK = NIDX // 128
PAD = NIDX + LANES

_mesh = plsc.VectorSubcoreMesh(core_axis_name="c", subcore_axis_name="s")

_T_BLK_C = 8192
_T_BLK_R = _T_BLK_C // 4


def _transpose_body(in_ref, out_ref):
    x = in_ref[...]
    eye = jnp.eye(EMB, dtype=jnp.float32)
    y = lax.dot_general(x, eye, (((0,), (0,)), ((), ())),
                        preferred_element_type=jnp.float32)
    y3 = y.reshape(_T_BLK_R, 4, EMB)
    for q in range(4):
        out_ref[:, q * EMB:(q + 1) * EMB] = y3[:, q, :]


def _emb_to_lin128(femb_t):
    grid = (100000 + _T_BLK_C - 1) // _T_BLK_C
    return pl.pallas_call(
        _transpose_body,
        grid=(grid,),
        in_specs=[pl.BlockSpec((32, _T_BLK_C), lambda j: (0, j))],
        out_specs=pl.BlockSpec((_T_BLK_R, 128), lambda j: (j, 0)),
        out_shape=jax.ShapeDtypeStruct((25000, 128), jnp.float32),
    )(femb_t)


@functools.partial(
    pl.kernel,
    mesh=_mesh,
    out_type=jax.ShapeDtypeStruct((BATCH,), jnp.float32),
    scratch_types=[
        pltpu.VMEM((NIDX,), jnp.int32),
        pltpu.VMEM((PAD,), jnp.float32),
        pltpu.VMEM((PAD,), jnp.float32),
        pltpu.VMEM((NIDX, EMB), jnp.float32),
        pltpu.VMEM((BPW, LANES), jnp.float32),
        pltpu.VMEM((BPW,), jnp.float32),
        pltpu.VMEM((LANES,), jnp.float32),
        pltpu.SemaphoreType.DMA,
    ],
    compiler_params=pltpu.CompilerParams(
        needs_layout_passes=False, use_tc_tiling_on_sc=False),
)
def _fm_sc(emb_hbm, fw_hbm, idx_hbm, fv_hbm, bias_hbm, out_hbm,
           idx_v, fv_v, fw_v, rows_v, vsum_v, out_v, bias_v, sem):
    wid = lax.axis_index("s") * NUM_CORES + lax.axis_index("c")
    base = wid * NIDX

    pltpu.sync_copy(idx_hbm.at[pl.ds(base, NIDX)], idx_v)
    pltpu.sync_copy(fv_hbm.at[pl.ds(base, NIDX)], fv_v.at[pl.ds(0, NIDX)])
    pltpu.sync_copy(bias_hbm, bias_v)

    copies = []
    for c in range(NCHUNK):
        sl = pl.ds(c * 128, 128)
        copies.append(
            pltpu.async_copy(emb_hbm.at[idx_v.at[sl]], rows_v.at[sl], sem))
        copies.append(
            pltpu.async_copy(fw_hbm.at[idx_v.at[sl]], fw_v.at[sl], sem))
    for cp in copies:
        cp.wait()

    iota = lax.iota(jnp.int32, LANES)
    m10 = iota < (NUM_FIELD - LANES)
    zeros = jnp.zeros((LANES,), jnp.float32)

    def row_body(b, carry):
        j0 = b * NUM_FIELD
        acc0 = acc1 = sq0 = sq1 = zeros
        fvr0 = fv_v[pl.ds(j0, LANES)]
        fvr1 = fv_v[pl.ds(j0 + LANES, LANES)]
        for f in range(NUM_FIELD):
            e0 = rows_v[j0 + f, pl.ds(0, LANES)]
            e1 = rows_v[j0 + f, pl.ds(LANES, LANES)]
            fvs = fvr0[f] if f < LANES else fvr1[f - LANES]
            t0 = e0 * fvs
            t1 = e1 * fvs
            acc0 = acc0 + t0
            acc1 = acc1 + t1
            sq0 = sq0 + t0 * t0
            sq1 = sq1 + t1 * t1
        v = (acc0 * acc0 + acc1 * acc1 - sq0 - sq1) * 0.5
        i0 = j0 + iota
        i1 = i0 + LANES
        p0 = plsc.load_gather(fv_v, [i0]) * plsc.load_gather(fw_v, [i0])
        p1 = plsc.load_gather(fv_v, [i1]) * plsc.load_gather(fw_v, [i1])
        v = v + p0 + jnp.where(m10, p1, 0.0)
        vsum_v[b, pl.ds(0, LANES)] = v
        return carry

    lax.fori_loop(0, BPW, row_body, 0)

    bias_vec = bias_v[...]

    def red_body(g, carry):
        rb = g * LANES + iota
        y = zeros
        for k in range(LANES):
            col = jnp.full((LANES,), k, jnp.int32)
            y = y + plsc.load_gather(vsum_v, [rb, col])
        x = y + bias_vec
        out_v[pl.ds(g * LANES, LANES)] = 1.0 / (1.0 + jnp.exp(-x))
        return carry

    lax.fori_loop(0, BPW // LANES, red_body, 0)

    pltpu.sync_copy(out_v, out_hbm.at[pl.ds(wid * BPW, BPW)])


def kernel(feat_index, feat_value, first_weights, feat_embeddings, bias):
    idx = feat_index.astype(jnp.int32).reshape(-1)
    fv = feat_value.astype(jnp.float32).reshape(-1)
    fw = first_weights.astype(jnp.float32).reshape(-1)
    bias_arr = jnp.full((LANES,), bias, jnp.float32)
    emb_lin = _emb_to_lin128(feat_embeddings.T).reshape(-1).reshape(100000, EMB)
    out = _fm_sc(emb_lin, fw, idx, fv, bias_arr)
    return out.reshape(BATCH, 1)


# speedup vs baseline: 1.6895x; 1.0259x over previous
import functools

import jax
import jax.numpy as jnp
from jax import lax
from jax.experimental import pallas as pl
from jax.experimental.pallas import tpu as pltpu
from jax.experimental.pallas import tpu_sc as plsc

BATCH = 4096
NUM_FIELD = 26
EMB = 32
LANES = 16

NUM_CORES = 2
NUM_SUBCORES = 16
NUM_WORKERS = NUM_CORES * NUM_SUBCORES
BPW = BATCH // NUM_WORKERS
NIDX = BPW * NUM_FIELD
NCHUNK = NIDX // 128
PAD = NIDX + LANES

_mesh = plsc.VectorSubcoreMesh(core_axis_name="c", subcore_axis_name="s")

_T_BLK_C = 8192
_T_BLK_R = _T_BLK_C // 4


def _transpose_body(in_ref, out_ref):
    x = in_ref[...]
    y = jnp.transpose(x)
    y3 = y.reshape(_T_BLK_R, 4, EMB)
    for q in range(4):
        out_ref[:, q * EMB:(q + 1) * EMB] = y3[:, q, :]


def _emb_to_lin128(femb_t):
    grid = (100000 + _T_BLK_C - 1) // _T_BLK_C
    return pl.pallas_call(
        _transpose_body,
        grid=(grid,),
        in_specs=[pl.BlockSpec((32, _T_BLK_C), lambda j: (0, j))],
        out_specs=pl.BlockSpec((_T_BLK_R, 128), lambda j: (j, 0)),
        out_shape=jax.ShapeDtypeStruct((25000, 128), jnp.float32),
    )(femb_t)


@functools.partial(
    pl.kernel,
    mesh=_mesh,
    out_type=jax.ShapeDtypeStruct((BATCH,), jnp.float32),
    scratch_types=[
        pltpu.VMEM((NIDX,), jnp.int32),
        pltpu.VMEM((PAD,), jnp.float32),
        pltpu.VMEM((PAD,), jnp.float32),
        pltpu.VMEM((NIDX, EMB), jnp.float32),
        pltpu.VMEM((BPW, LANES), jnp.float32),
        pltpu.VMEM((BPW,), jnp.float32),
        pltpu.VMEM((LANES,), jnp.float32),
        pltpu.SemaphoreType.DMA,
    ],
    compiler_params=pltpu.CompilerParams(
        needs_layout_passes=False, use_tc_tiling_on_sc=False),
)
def _fm_sc(emb_hbm, fw_hbm, idx_hbm, fv_hbm, bias_hbm, out_hbm,
           idx_v, fv_v, fw_v, rows_v, vsum_v, out_v, bias_v, sem):
    wid = lax.axis_index("s") * NUM_CORES + lax.axis_index("c")
    base = wid * NIDX

    pltpu.sync_copy(idx_hbm.at[pl.ds(base, NIDX)], idx_v)
    pltpu.sync_copy(fv_hbm.at[pl.ds(base, NIDX)], fv_v.at[pl.ds(0, NIDX)])
    pltpu.sync_copy(bias_hbm, bias_v)

    copies = []
    for c in range(NCHUNK):
        sl = pl.ds(c * 128, 128)
        copies.append(
            pltpu.async_copy(emb_hbm.at[idx_v.at[sl]], rows_v.at[sl], sem))
        copies.append(
            pltpu.async_copy(fw_hbm.at[idx_v.at[sl]], fw_v.at[sl], sem))
    for cp in copies:
        cp.wait()

    iota = lax.iota(jnp.int32, LANES)
    m10 = iota < (NUM_FIELD - LANES)
    zeros = jnp.zeros((LANES,), jnp.float32)

    def row_body(b, carry):
        j0 = b * NUM_FIELD
        acc0 = acc1 = sq0 = sq1 = zeros
        fvr0 = fv_v[pl.ds(j0, LANES)]
        fvr1 = fv_v[pl.ds(j0 + LANES, LANES)]
        for f in range(NUM_FIELD):
            e0 = rows_v[j0 + f, pl.ds(0, LANES)]
            e1 = rows_v[j0 + f, pl.ds(LANES, LANES)]
            fvs = fvr0[f] if f < LANES else fvr1[f - LANES]
            t0 = e0 * fvs
            t1 = e1 * fvs
            acc0 = acc0 + t0
            acc1 = acc1 + t1
            sq0 = sq0 + t0 * t0
            sq1 = sq1 + t1 * t1
        v = (acc0 * acc0 + acc1 * acc1 - sq0 - sq1) * 0.5
        i0 = j0 + iota
        i1 = i0 + LANES
        p0 = plsc.load_gather(fv_v, [i0]) * plsc.load_gather(fw_v, [i0])
        p1 = plsc.load_gather(fv_v, [i1]) * plsc.load_gather(fw_v, [i1])
        v = v + p0 + jnp.where(m10, p1, 0.0)
        vsum_v[b, pl.ds(0, LANES)] = v
        return carry

    lax.fori_loop(0, BPW, row_body, 0)

    bias_vec = bias_v[...]

    def red_body(g, carry):
        rb = g * LANES + iota
        y = zeros
        for k in range(LANES):
            col = jnp.full((LANES,), k, jnp.int32)
            y = y + plsc.load_gather(vsum_v, [rb, col])
        x = y + bias_vec
        out_v[pl.ds(g * LANES, LANES)] = 1.0 / (1.0 + jnp.exp(-x))
        return carry

    lax.fori_loop(0, BPW // LANES, red_body, 0)

    pltpu.sync_copy(out_v, out_hbm.at[pl.ds(wid * BPW, BPW)])


def kernel(feat_index, feat_value, first_weights, feat_embeddings, bias):
    idx = feat_index.astype(jnp.int32).reshape(-1)
    fv = feat_value.astype(jnp.float32).reshape(-1)
    fw = first_weights.astype(jnp.float32).reshape(-1)
    bias_arr = jnp.full((LANES,), bias, jnp.float32)
    emb_lin = _emb_to_lin128(feat_embeddings.T).reshape(-1).reshape(100000, EMB)
    out = _fm_sc(emb_lin, fw, idx, fv, bias_arr)
    return out.reshape(BATCH, 1)
